# baseline (speedup 1.0000x reference)
import jax
import jax.numpy as jnp
from jax import lax
from jax.experimental import pallas as pl
from jax.experimental.pallas import tpu as pltpu
from jax.experimental.pallas import tpu_sc as plsc

_T = 8192
_D = 768


def _make_sc_copy():
    mesh = plsc.VectorSubcoreMesh(core_axis_name="c", subcore_axis_name="s")
    num_workers = 32
    rows_per_w = _T // num_workers

    def body(wpe_hbm, out_hbm):
        wid = lax.axis_index("s") * 2 + lax.axis_index("c")
        base = wid * rows_per_w
        pltpu.sync_copy(
            wpe_hbm.at[pl.ds(base, rows_per_w)],
            out_hbm.at[pl.ds(base, rows_per_w)],
        )

    return pl.kernel(
        body,
        out_type=jax.ShapeDtypeStruct((_T, _D), jnp.float32),
        mesh=mesh,
    )


_sc_copy = _make_sc_copy()


def kernel(idx, wpe):
    del idx
    return _sc_copy(wpe)


# speedup vs baseline: 21.7412x; 21.7412x over previous
import jax
import jax.numpy as jnp
from jax import lax
from jax.experimental import pallas as pl
from jax.experimental.pallas import tpu as pltpu
from jax.experimental.pallas import tpu_sc as plsc

_T = 8192
_D = 768
_NW = 32
_RPW = _T // _NW
_CH = 32
_NCH = _RPW // _CH
_NBUF = 4


def _make_sc_copy():
    mesh = plsc.VectorSubcoreMesh(core_axis_name="c", subcore_axis_name="s")

    def body(wpe_hbm, out_hbm, *scratch):
        bufs = scratch[:_NBUF]
        rsems = scratch[_NBUF:2 * _NBUF]
        wsems = scratch[2 * _NBUF:3 * _NBUF]
        wid = lax.axis_index("s") * 2 + lax.axis_index("c")
        base = wid * _RPW

        def rd(i):
            return pltpu.async_copy(
                wpe_hbm.at[pl.ds(base + i * _CH, _CH)],
                bufs[i % _NBUF], rsems[i % _NBUF])

        def wr(i):
            return pltpu.async_copy(
                bufs[i % _NBUF],
                out_hbm.at[pl.ds(base + i * _CH, _CH)], wsems[i % _NBUF])

        reads = {j: rd(j) for j in range(_NBUF)}
        writes = {}
        for i in range(_NCH):
            reads[i].wait()
            writes[i] = wr(i)
            nxt = i + _NBUF
            if nxt < _NCH:
                writes[i].wait()
                reads[nxt] = rd(nxt)
        for i in range(max(0, _NCH - _NBUF), _NCH):
            writes[i].wait()

    return pl.kernel(
        body,
        out_type=jax.ShapeDtypeStruct((_T, _D), jnp.float32),
        mesh=mesh,
        scratch_types=(
            [pltpu.VMEM((_CH, _D), jnp.float32) for _ in range(_NBUF)]
            + [pltpu.SemaphoreType.DMA for _ in range(2 * _NBUF)]
        ),
    )


_sc_copy = _make_sc_copy()


def kernel(idx, wpe):
    del idx
    return _sc_copy(wpe)
